# Initial kernel scaffold; baseline (speedup 1.0000x reference)
import functools

import jax
import jax.numpy as jnp
from jax import lax
from jax.experimental import pallas as pl
from jax.experimental.pallas import tpu as pltpu
from jax.experimental.pallas import tpu_sc as plsc

_N = 10000
_E = 160000
_NTILE = 16
_LANES = 16
_ROWS = 79
_EPT = _ROWS * 128
_NP = 10240
_SLICE = _NP // _NTILE
_TRASH = _NP - 1
_ROUNDS = 4


def _rsqrt16(x):
    i = plsc.bitcast(x, jnp.int32)
    i = jnp.int32(0x5F3759DF) - lax.shift_right_arithmetic(i, 1)
    y = plsc.bitcast(i, jnp.float32)
    for _ in range(3):
        y = y * (1.5 - 0.5 * x * y * y)
    return y


def _sc_body(src_hbm, dst_hbm, out_hbm, src_v, dst_v, w_v, vals_v, ubuf_v,
             slice_v, a_sh, b_sh):
    s = lax.axis_index("s")
    c = lax.axis_index("c")
    base = s * _SLICE

    pltpu.sync_copy(src_hbm.at[s], src_v)
    pltpu.sync_copy(dst_hbm.at[s], dst_v)

    zero16 = jnp.zeros((_LANES,), jnp.float32)
    one16 = jnp.full((_LANES,), 1.0, jnp.float32)

    def _zero_slice(i, _):
        slice_v[pl.ds(i * _LANES, _LANES)] = zero16
        return _

    lax.fori_loop(0, _SLICE // _LANES, _zero_slice, None)
    pltpu.sync_copy(slice_v, a_sh.at[pl.ds(base, _SLICE)])

    def _fill_ones(j, _):
        for k in range(8):
            vals_v[j, pl.ds(k * _LANES, _LANES)] = one16
        return _

    lax.fori_loop(0, _ROWS, _fill_ones, None)
    plsc.subcore_barrier()

    def _deg_row(j, _):
        pltpu.sync_copy(vals_v.at[j], a_sh.at[dst_v.at[j]], add=True)
        return _

    lax.fori_loop(0, _ROWS, _deg_row, None)
    plsc.subcore_barrier()

    pltpu.sync_copy(a_sh.at[pl.ds(base, _SLICE)], slice_v)

    def _norm_step(i, _):
        d = slice_v[pl.ds(i * _LANES, _LANES)]
        slice_v[pl.ds(i * _LANES, _LANES)] = _rsqrt16(jnp.maximum(d, 1.0))
        return _

    lax.fori_loop(0, _SLICE // _LANES, _norm_step, None)
    pltpu.sync_copy(slice_v, b_sh.at[pl.ds(base, _SLICE)])
    plsc.subcore_barrier()

    pltpu.sync_copy(b_sh, ubuf_v)
    plsc.subcore_barrier()

    def _w_row(j, _):
        for k in range(8):
            sl = pl.ds(k * _LANES, _LANES)
            ns = plsc.load_gather(ubuf_v, [src_v[j, sl]])
            nd = plsc.load_gather(ubuf_v, [dst_v[j, sl]])
            w_v[j, sl] = ns * nd
        return _

    lax.fori_loop(0, _ROWS, _w_row, None)

    inv_n = jnp.float32(1.0 / _N)

    def _u0_step(i, _):
        g = base + i * _LANES + lax.iota(jnp.int32, _LANES)
        slice_v[pl.ds(i * _LANES, _LANES)] = jnp.where(g < _N, inv_n, 0.0)
        return _

    lax.fori_loop(0, _SLICE // _LANES, _u0_step, None)
    pltpu.sync_copy(slice_v, a_sh.at[pl.ds(base, _SLICE)])
    lax.fori_loop(0, _SLICE // _LANES, _zero_slice, None)
    pltpu.sync_copy(slice_v, b_sh.at[pl.ds(base, _SLICE)])
    plsc.subcore_barrier()

    for r in range(_ROUNDS):
        rd, wr = (a_sh, b_sh) if r % 2 == 0 else (b_sh, a_sh)
        pltpu.sync_copy(rd, ubuf_v)
        lax.fori_loop(0, _SLICE // _LANES, _zero_slice, None)
        pltpu.sync_copy(slice_v, wr.at[pl.ds(base, _SLICE)])
        plsc.subcore_barrier()

        def _mp_row(j, _):
            for k in range(8):
                sl = pl.ds(k * _LANES, _LANES)
                uv = plsc.load_gather(ubuf_v, [dst_v[j, sl]])
                vals_v[j, sl] = uv * w_v[j, sl]
            return _

        lax.fori_loop(0, _ROWS, _mp_row, None)

        def _scat_row(j, _):
            pltpu.sync_copy(vals_v.at[j], wr.at[src_v.at[j]], add=True)
            return _

        lax.fori_loop(0, _ROWS, _scat_row, None)
        plsc.subcore_barrier()

    pltpu.sync_copy(a_sh.at[pl.ds(base, _SLICE)], slice_v)

    @pl.when(c == 0)
    def _():
        pltpu.sync_copy(slice_v, out_hbm.at[pl.ds(base, _SLICE)])


_sc_u = functools.partial(
    pl.kernel,
    out_type=jax.ShapeDtypeStruct((_NP,), jnp.float32),
    mesh=plsc.VectorSubcoreMesh(core_axis_name="c", subcore_axis_name="s"),
    scratch_types=[
        pltpu.VMEM((_ROWS, 128), jnp.int32),
        pltpu.VMEM((_ROWS, 128), jnp.int32),
        pltpu.VMEM((_ROWS, 128), jnp.float32),
        pltpu.VMEM((_ROWS, 128), jnp.float32),
        pltpu.VMEM((_NP,), jnp.float32),
        pltpu.VMEM((_SLICE,), jnp.float32),
        pltpu.VMEM_SHARED((_NP,), jnp.float32),
        pltpu.VMEM_SHARED((_NP,), jnp.float32),
    ],
)(_sc_body)


def _tc_body(u_ref, feat_ref, wemb_ref, bemb_ref, w1_ref, b1_ref, w2_ref,
             b2_ref, wr_ref, br_ref, out_ref):
    u = u_ref[...]
    dn = (((1,), (1,)), ((), ()))
    v1 = lax.dot_general(u, feat_ref[...], (((1,), (0,)), ((), ())),
                         preferred_element_type=jnp.float32)
    su = jnp.sum(u)
    hg = lax.dot_general(v1, wemb_ref[...], dn,
                         preferred_element_type=jnp.float32) + su * bemb_ref[...]
    hg = lax.dot_general(hg, w1_ref[...], dn,
                         preferred_element_type=jnp.float32) + b1_ref[...]
    hg = lax.dot_general(hg, w2_ref[...], dn,
                         preferred_element_type=jnp.float32) + b2_ref[...]
    out_ref[...] = lax.dot_general(hg, wr_ref[...], dn,
                                   preferred_element_type=jnp.float32) + br_ref[...]


def kernel(feat, edge_index, e, snorm_n, snorm_e, W_emb, b_emb, W1, b1, W2,
           b2, Wr, br):
    src = edge_index[0]
    dst = edge_index[1]
    pad = _NTILE * _EPT - _E
    fill = jnp.full((pad,), _TRASH, jnp.int32)
    srcp = jnp.concatenate([src, fill]).reshape(_NTILE, _ROWS, 128)
    dstp = jnp.concatenate([dst, fill]).reshape(_NTILE, _ROWS, 128)

    u = _sc_u(srcp, dstp)
    u2 = u[:_N].reshape(1, _N)

    nc = Wr.shape[0]
    out = pl.pallas_call(
        _tc_body,
        out_shape=jax.ShapeDtypeStruct((1, nc), jnp.float32),
    )(u2, feat, W_emb, b_emb.reshape(1, -1), W1, b1.reshape(1, -1), W2,
      b2.reshape(1, -1), Wr, br.reshape(1, -1))
    return out


# speedup vs baseline: 66.1688x; 66.1688x over previous
import functools

import jax
import jax.numpy as jnp
from jax import lax
from jax.experimental import pallas as pl
from jax.experimental.pallas import tpu as pltpu
from jax.experimental.pallas import tpu_sc as plsc

_N = 10000
_E = 160000
_NTILE = 16
_LANES = 16
_ROWS = 79
_EPT = _ROWS * 128
_NP = 10240
_SLICE = _NP // _NTILE
_TRASH = _NP - 1
_ROUNDS = 4


def _rsqrt16(x):
    i = lax.bitcast_convert_type(x, jnp.int32)
    i = jnp.int32(0x5F3759DF) - lax.shift_right_arithmetic(i, 1)
    y = lax.bitcast_convert_type(i, jnp.float32)
    for _ in range(3):
        y = y * (1.5 - 0.5 * x * y * y)
    return y


def _sc_body(src_hbm, dst_hbm, out_hbm, src_v, dst_v, w_v, vals_v, ubuf_v,
             slice_v, a_sh, b_sh):
    s = lax.axis_index("s")
    c = lax.axis_index("c")
    base = s * _SLICE

    pltpu.sync_copy(src_hbm.at[s], src_v)
    pltpu.sync_copy(dst_hbm.at[s], dst_v)

    zero16 = jnp.zeros((_LANES,), jnp.float32)
    one16 = jnp.full((_LANES,), 1.0, jnp.float32)

    def _zero_slice(i, _):
        slice_v[pl.ds(i * _LANES, _LANES)] = zero16
        return _

    lax.fori_loop(0, _SLICE // _LANES, _zero_slice, None)
    pltpu.sync_copy(slice_v, a_sh.at[pl.ds(base, _SLICE)])

    def _fill_ones(j, _):
        for k in range(8):
            vals_v[j, pl.ds(k * _LANES, _LANES)] = one16
        return _

    lax.fori_loop(0, _ROWS, _fill_ones, None)
    plsc.subcore_barrier()

    def _deg_row(j, _):
        pltpu.sync_copy(vals_v.at[j], a_sh.at[dst_v.at[j]], add=True)
        return _

    lax.fori_loop(0, _ROWS, _deg_row, None)
    plsc.subcore_barrier()

    pltpu.sync_copy(a_sh.at[pl.ds(base, _SLICE)], slice_v)

    def _norm_step(i, _):
        d = slice_v[pl.ds(i * _LANES, _LANES)]
        slice_v[pl.ds(i * _LANES, _LANES)] = _rsqrt16(jnp.maximum(d, 1.0))
        return _

    lax.fori_loop(0, _SLICE // _LANES, _norm_step, None)
    pltpu.sync_copy(slice_v, b_sh.at[pl.ds(base, _SLICE)])
    plsc.subcore_barrier()

    pltpu.sync_copy(b_sh, ubuf_v)
    plsc.subcore_barrier()

    def _w_row(j, _):
        for k in range(8):
            sl = pl.ds(k * _LANES, _LANES)
            ns = plsc.load_gather(ubuf_v, [src_v[j, sl]])
            nd = plsc.load_gather(ubuf_v, [dst_v[j, sl]])
            w_v[j, sl] = ns * nd
        return _

    lax.fori_loop(0, _ROWS, _w_row, None)

    inv_n = jnp.float32(1.0 / _N)

    def _u0_step(i, _):
        g = base + i * _LANES + lax.iota(jnp.int32, _LANES)
        slice_v[pl.ds(i * _LANES, _LANES)] = jnp.where(g < _N, inv_n, 0.0)
        return _

    lax.fori_loop(0, _SLICE // _LANES, _u0_step, None)
    pltpu.sync_copy(slice_v, a_sh.at[pl.ds(base, _SLICE)])
    lax.fori_loop(0, _SLICE // _LANES, _zero_slice, None)
    pltpu.sync_copy(slice_v, b_sh.at[pl.ds(base, _SLICE)])
    plsc.subcore_barrier()

    for r in range(_ROUNDS):
        rd, wr = (a_sh, b_sh) if r % 2 == 0 else (b_sh, a_sh)
        pltpu.sync_copy(rd, ubuf_v)
        lax.fori_loop(0, _SLICE // _LANES, _zero_slice, None)
        pltpu.sync_copy(slice_v, wr.at[pl.ds(base, _SLICE)])
        plsc.subcore_barrier()

        def _mp_row(j, _):
            for k in range(8):
                sl = pl.ds(k * _LANES, _LANES)
                uv = plsc.load_gather(ubuf_v, [dst_v[j, sl]])
                vals_v[j, sl] = uv * w_v[j, sl]
            return _

        lax.fori_loop(0, _ROWS, _mp_row, None)

        def _scat_row(j, _):
            pltpu.sync_copy(vals_v.at[j], wr.at[src_v.at[j]], add=True)
            return _

        lax.fori_loop(0, _ROWS, _scat_row, None)
        plsc.subcore_barrier()

    pltpu.sync_copy(a_sh.at[pl.ds(base, _SLICE)], slice_v)

    @pl.when(c == 0)
    def _():
        pltpu.sync_copy(slice_v, out_hbm.at[pl.ds(base, _SLICE)])


_sc_u = functools.partial(
    pl.kernel,
    out_type=jax.ShapeDtypeStruct((_NP,), jnp.float32),
    mesh=plsc.VectorSubcoreMesh(core_axis_name="c", subcore_axis_name="s"),
    compiler_params=pltpu.CompilerParams(needs_layout_passes=False),
    scratch_types=[
        pltpu.VMEM((_ROWS, 128), jnp.int32),
        pltpu.VMEM((_ROWS, 128), jnp.int32),
        pltpu.VMEM((_ROWS, 128), jnp.float32),
        pltpu.VMEM((_ROWS, 128), jnp.float32),
        pltpu.VMEM((_NP,), jnp.float32),
        pltpu.VMEM((_SLICE,), jnp.float32),
        pltpu.VMEM_SHARED((_NP,), jnp.float32),
        pltpu.VMEM_SHARED((_NP,), jnp.float32),
    ],
)(_sc_body)


def _tc_body(u_ref, feat_ref, wemb_ref, bemb_ref, w1_ref, b1_ref, w2_ref,
             b2_ref, wr_ref, br_ref, out_ref):
    u = u_ref[...]
    dn = (((1,), (1,)), ((), ()))
    v1 = lax.dot_general(u, feat_ref[...], (((1,), (0,)), ((), ())),
                         preferred_element_type=jnp.float32)
    su = jnp.sum(u)
    hg = lax.dot_general(v1, wemb_ref[...], dn,
                         preferred_element_type=jnp.float32) + su * bemb_ref[...]
    hg = lax.dot_general(hg, w1_ref[...], dn,
                         preferred_element_type=jnp.float32) + b1_ref[...]
    hg = lax.dot_general(hg, w2_ref[...], dn,
                         preferred_element_type=jnp.float32) + b2_ref[...]
    out_ref[...] = lax.dot_general(hg, wr_ref[...], dn,
                                   preferred_element_type=jnp.float32) + br_ref[...]


def kernel(feat, edge_index, e, snorm_n, snorm_e, W_emb, b_emb, W1, b1, W2,
           b2, Wr, br):
    src = edge_index[0]
    dst = edge_index[1]
    pad = _NTILE * _EPT - _E
    fill = jnp.full((pad,), _TRASH, jnp.int32)
    srcp = jnp.concatenate([src, fill]).reshape(_NTILE, _ROWS, 128)
    dstp = jnp.concatenate([dst, fill]).reshape(_NTILE, _ROWS, 128)

    u = _sc_u(srcp, dstp)
    u2 = u[:_N].reshape(1, _N)

    nc = Wr.shape[0]
    out = pl.pallas_call(
        _tc_body,
        out_shape=jax.ShapeDtypeStruct((1, nc), jnp.float32),
    )(u2, feat, W_emb, b_emb.reshape(1, -1), W1, b1.reshape(1, -1), W2,
      b2.reshape(1, -1), Wr, br.reshape(1, -1))
    return out
